# baseline (speedup 1.0000x reference)
import functools

import jax
import jax.numpy as jnp
from jax import lax
from jax.experimental import pallas as pl
from jax.experimental.pallas import tpu as pltpu
from jax.experimental.pallas import tpu_sc as plsc

N = 50000
E = 800000
NODE_DIM = 64
EDGE_DIM = 32
NUM_LAYERS = 3
NUM_ROBOTS = 2

DG = 16
GW = 1280
PK = 8
R = 4000
NCP = E // PK // R
CHN = 5000
PD = PK * NODE_DIM


def _sc_gather(table, idx):
    mesh = plsc.VectorSubcoreMesh(core_axis_name="core",
                                  subcore_axis_name="subcore")

    @functools.partial(
        pl.kernel,
        out_type=jax.ShapeDtypeStruct((E, DG), jnp.float32),
        mesh=mesh,
        compiler_params=pltpu.CompilerParams(use_tc_tiling_on_sc=False),
    )
    def gk(x_hbm, i_hbm, o_hbm):
        def body(i_vmem, o_vmem):
            pltpu.sync_copy(x_hbm.at[i_vmem.at[0]], o_vmem)

        pltpu.emit_pipeline(
            body,
            grid=(E // GW,),
            in_specs=[pl.BlockSpec((1, GW), index_map=lambda i: (0, i))],
            out_specs=[pl.BlockSpec((GW, DG), index_map=lambda i: (i, 0))],
            core_axis_name=("core", "subcore"),
            dimension_semantics=(pltpu.PARALLEL,),
        )(i_hbm, o_hbm)

    return gk(table, idx.reshape(1, E))


def _dot(a, b):
    f32 = jnp.float32
    bf = jnp.bfloat16
    ahi = a.astype(bf)
    alo = (a - ahi.astype(f32)).astype(bf)
    bhi = b.astype(bf)
    blo = (b - bhi.astype(f32)).astype(bf)
    dn = (((1,), (0,)), ((), ()))
    dot = lambda x, y: jax.lax.dot_general(x, y, dn,
                                           preferred_element_type=f32)
    return dot(alo, bhi) + dot(ahi, blo) + dot(ahi, bhi)


def _dot_fast(a, b):
    return jax.lax.dot_general(a.astype(jnp.bfloat16), b,
                               (((1,), (0,)), ((), ())),
                               preferred_element_type=jnp.float32)


def hdot0(a, b):
    return jax.lax.dot_general(a, b, (((1,), (0,)), ((), ())),
                               precision=jax.lax.Precision.HIGHEST)


def _bdiag(w, reps):
    r, c = w.shape
    out = jnp.zeros((r * reps, c * reps), w.dtype)
    for j in range(reps):
        out = out.at[j * r:(j + 1) * r, j * c:(j + 1) * c].set(w)
    return out


def _fold(v):
    t = v[0:1, 0:NODE_DIM]
    for j in range(1, PK):
        t = t + v[0:1, j * NODE_DIM:(j + 1) * NODE_DIM]
    return t


def _edge_pass_body(gp_ref, efp_ref, ones_row, nw1bd, nb1t, ew1bd, eb1t,
                    onesbd, abd, bbd, dconst, wt, out_ref,
                    acc, cacc, dt):
    l = pl.program_id(0)
    c = pl.program_id(1)

    @pl.when(c == 0)
    def _():
        @pl.when(l == 0)
        def _():
            cacc[...] = jnp.zeros_like(cacc)

        @pl.when(l > 0)
        def _():
            cacc[...] = cacc[...] + _fold(acc[...]) * (1.0 / E)

        acc[...] = jnp.zeros_like(acc)
        d = dconst[0] + _dot(cacc[...], wt[0])
        dt[...] = jnp.concatenate([d] * PK, axis=1)

    def dot2(x, b):
        hi = x.astype(jnp.bfloat16)
        lo = (x - hi.astype(jnp.float32)).astype(jnp.bfloat16)
        return _dot_fast(hi, b) + _dot_fast(lo, b)

    r1 = jax.nn.relu(_dot_fast(gp_ref[...], nw1bd[...]) + nb1t[...])
    r2 = jax.nn.relu(_dot_fast(efp_ref[...], ew1bd[...]) + eb1t[...])
    u = dot2(r1, abd[0]) + _dot_fast(r2, bbd[0]) + dt[...]
    h = jax.nn.relu(u)
    mu = dot2(h, onesbd[...]) * (1.0 / NODE_DIM)
    dlt = h - mu
    var = dot2(dlt * dlt, onesbd[...]) * (1.0 / NODE_DIM)
    v = var + 1e-5
    r0 = jax.lax.rsqrt(v)
    rs = r0 * (1.5 - 0.5 * v * r0 * r0)
    m = dlt * rs
    mhi = m.astype(jnp.bfloat16)
    mlo = (m - mhi.astype(jnp.float32)).astype(jnp.bfloat16)
    acc[...] = (acc[...] + _dot_fast(ones_row[...], mhi)
                + _dot_fast(ones_row[...], mlo))

    @pl.when((l == NUM_LAYERS - 1) & (c == NCP - 1))
    def _():
        out_ref[...] = cacc[...] + _fold(acc[...]) * (1.0 / E)


def _edge_pass(gp, efp, ones_row, nw1bd, nb1t, ew1bd, eb1t, onesbd,
               abd, bbd, dconst, wt):
    full2 = lambda arr: pl.BlockSpec(arr.shape, lambda l, c: (0, 0))
    per_layer = lambda arr: pl.BlockSpec((1,) + arr.shape[1:],
                                         lambda l, c: (l, 0, 0))
    return pl.pallas_call(
        _edge_pass_body,
        grid=(NUM_LAYERS, NCP),
        in_specs=[
            pl.BlockSpec((R, PK * DG), lambda l, c: (c, 0)),
            pl.BlockSpec((R, PK * 3), lambda l, c: (c, 0)),
            full2(ones_row),
            full2(nw1bd), full2(nb1t), full2(ew1bd), full2(eb1t),
            full2(onesbd),
            per_layer(abd), per_layer(bbd), per_layer(dconst),
            per_layer(wt),
        ],
        out_specs=pl.BlockSpec((1, NODE_DIM), lambda l, c: (0, 0)),
        out_shape=jax.ShapeDtypeStruct((1, NODE_DIM), jnp.float32),
        scratch_shapes=[
            pltpu.VMEM((1, PD), jnp.float32),
            pltpu.VMEM((1, NODE_DIM), jnp.float32),
            pltpu.VMEM((1, PD), jnp.float32),
        ],
    )(gp, efp, ones_row, nw1bd, nb1t, ew1bd, eb1t, onesbd,
      abd, bbd, dconst, wt)


def _affinity_body(nf_ref, ct, nw1, nb1, nw2, nb2, w1r, w1f, w1f2, bc, w2p,
                   ab2, out_ref, rpre, base):
    c = pl.program_id(0)

    def node_mlp(x):
        return _dot(jax.nn.relu(_dot(x, nw1[...]) + nb1[...]),
                    nw2[...]) + nb2[...]

    @pl.when(c == 0)
    def _():
        z8 = node_mlp(nf_ref[0:8, :]) + ct[...]
        rpre[...] = _dot(z8, w1r[...])
        base[...] = bc[...] + _dot(ct[...], w1f[...])

    r = jax.nn.relu(_dot(nf_ref[...], nw1[...]) + nb1[...])
    f = _dot(r, w1f2[...]) + base[...]
    s0 = _dot(jax.nn.relu(f + rpre[0:1, :]), w2p[...]) + ab2[...]
    s1 = _dot(jax.nn.relu(f + rpre[1:2, :]), w2p[...]) + ab2[...]
    out_ref[...] = jnp.concatenate([s0[:, 0:1], s1[:, 0:1]], axis=1)


def _affinity(nf8, ct, nw1, nb1, nw2, nb2, w1r, w1f, w1f2, bc, w2p, ab2):
    full2 = lambda arr: pl.BlockSpec(arr.shape, lambda c: (0, 0))
    return pl.pallas_call(
        _affinity_body,
        grid=(N // CHN,),
        in_specs=[
            pl.BlockSpec((CHN, 8), lambda c: (c, 0)),
            full2(ct), full2(nw1), full2(nb1), full2(nw2), full2(nb2),
            full2(w1r), full2(w1f), full2(w1f2), full2(bc), full2(w2p),
            full2(ab2),
        ],
        out_specs=pl.BlockSpec((CHN, 2), lambda c: (c, 0)),
        out_shape=jax.ShapeDtypeStruct((N, 2), jnp.float32),
        scratch_shapes=[
            pltpu.VMEM((8, NODE_DIM), jnp.float32),
            pltpu.VMEM((1, NODE_DIM), jnp.float32),
        ],
    )(nf8, ct, nw1, nb1, nw2, nb2, w1r, w1f, w1f2, bc, w2p, ab2)


def kernel(node_features, edge_features, edge_indices, params):
    p = params
    bf = jnp.bfloat16
    src = edge_indices[:, 0]
    nf16 = jnp.pad(node_features, ((0, 0), (0, DG - 5)))

    g = _sc_gather(nf16, src)
    gp = g.reshape(E // PK, PK * DG)
    efp = edge_features.reshape(E // PK, PK * 3).astype(bf)

    nw1g = jnp.pad(p["ne_W1"], ((0, DG - 5), (0, 0)))
    nw1bd = _bdiag(nw1g, PK).astype(bf)
    nb1t = jnp.tile(p["ne_b1"].reshape(1, -1), (1, PK))
    ew1bd = _bdiag(p["ee_W1"], PK).astype(bf)
    eb1t = jnp.tile(p["ee_b1"].reshape(1, -1), (1, PK))
    onesbd = _bdiag(jnp.ones((NODE_DIM, NODE_DIM), jnp.float32),
                    PK).astype(bf)

    wt = jnp.stack([lp["W"][:NODE_DIM] for lp in p["layers"]])
    wb = jnp.stack([lp["W"][NODE_DIM:] for lp in p["layers"]])
    abd = jnp.stack([_bdiag(hdot0(p["ne_W2"], w), PK) for w in wt]).astype(bf)
    bbd = jnp.stack([_bdiag(hdot0(p["ee_W2"], w), PK) for w in wb]).astype(bf)
    dconst = jnp.stack([
        (p["ne_b2"].reshape(1, -1) @ wt[i] + p["ee_b2"].reshape(1, -1) @ wb[i]
         + lp["b"].reshape(1, -1))
        for i, lp in enumerate(p["layers"])])
    ones_row = jnp.ones((1, R), jnp.float32)

    ct = _edge_pass(gp, efp, ones_row, nw1bd, nb1t, ew1bd, eb1t, onesbd,
                    abd, bbd, dconst, wt)

    hdot = functools.partial(jax.lax.dot_general,
                             dimension_numbers=(((1,), (0,)), ((), ())),
                             precision=jax.lax.Precision.HIGHEST)
    nf8 = jnp.pad(node_features, ((0, 0), (0, 3)))
    nw1a = jnp.pad(p["ne_W1"], ((0, 3), (0, 0)))
    nb1 = p["ne_b1"].reshape(1, -1)
    nb2 = p["ne_b2"].reshape(1, -1)
    w1r = p["af_W1"][:NODE_DIM]
    w1f = p["af_W1"][NODE_DIM:]
    w1f2 = hdot(p["ne_W2"], w1f)
    bc = hdot(nb2, w1f) + p["af_b1"].reshape(1, -1)
    w2p = jnp.pad(p["af_W2"], ((0, 0), (0, 7)))
    ab2 = jnp.broadcast_to(p["af_b2"].reshape(1, 1), (1, 8))

    s = _affinity(nf8, ct, nw1a, nb1, p["ne_W2"], nb2,
                  w1r, w1f, w1f2, bc, w2p, ab2)
    return s[NUM_ROBOTS:, :].T
